# baseline (speedup 1.0000x reference)
import functools

import jax
import jax.numpy as jnp
from jax import lax
from jax.experimental import pallas as pl
from jax.experimental.pallas import tpu as pltpu
from jax.experimental.pallas import tpu_sc as plsc

_K = 1000
_KP = 1024
_D = 2048
_DH = _D // 2
_ALPHA = 10.0
_TOPM = 20

_B = 16384
_NTILES = 16
_ROWS_PER_TILE = _B // _NTILES
_CH = 32
_NCH = _ROWS_PER_TILE // _CH


@functools.partial(
    pl.kernel,
    mesh=plsc.VectorSubcoreMesh(core_axis_name="c", subcore_axis_name="s"),
    out_type=jax.ShapeDtypeStruct((_KP, 2, _DH), jnp.float32),
    scratch_types=[
        pltpu.VMEM((_CH, _CH), jnp.int32),
        pltpu.VMEM((_CH, 1, _DH), jnp.float32),
        pltpu.VMEM_SHARED((_KP, 1, _DH), jnp.float32),
        pltpu.SemaphoreType.DMA,
    ],
)
def _sc_segsum(y2_hbm, feats3_hbm, zeros_hbm, out_hbm, lab_v, stage_v,
               acc_sh, sem):
    cid = lax.axis_index("c")
    sid = lax.axis_index("s")

    @pl.when(sid == 0)
    def _zero():
        pltpu.sync_copy(zeros_hbm, acc_sh)

    plsc.subcore_barrier()

    base = sid * _ROWS_PER_TILE
    pltpu.sync_copy(
        y2_hbm.at[pl.ds(pl.multiple_of(base // _CH, _CH), _CH)], lab_v)

    def chunk(j, carry):
        row0 = pl.multiple_of(base + j * _CH, _CH)
        pltpu.async_copy(
            feats3_hbm.at[pl.ds(row0, _CH), pl.ds(cid, 1)], stage_v, sem
        ).wait()
        pltpu.sync_copy(stage_v, acc_sh.at[lab_v.at[j]], add=True)
        return carry

    lax.fori_loop(0, _NCH, chunk, jnp.int32(0))

    plsc.subcore_barrier()

    @pl.when(sid == 0)
    def _writeback():
        pltpu.sync_copy(acc_sh, out_hbm.at[:, pl.ds(cid, 1)])


def _loss_body(sums_ref, conf_ref, out_ref, key_ref):
    s = sums_ref[...]
    n2 = jnp.sum(s * s, axis=1, keepdims=True)
    inv = jnp.where(n2 > 0.0, jax.lax.rsqrt(jnp.maximum(n2, 1e-30)), 0.0)
    sn = (s * inv).astype(jnp.bfloat16)
    g = jax.lax.dot_general(
        sn, sn, (((1,), (1,)), ((), ())),
        preferred_element_type=jnp.float32)

    row_i = jax.lax.broadcasted_iota(jnp.int32, (_KP, _KP), 0)
    col_i = jax.lax.broadcasted_iota(jnp.int32, (_KP, _KP), 1)
    eye = row_i == col_i
    gd = jnp.where(eye, g, 0.0)
    zr = jnp.sum(gd, axis=1, keepdims=True)
    zc = jnp.sum(gd, axis=0, keepdims=True)
    d2 = zr + zc - 2.0 * g

    pr = (zr > 0.5).astype(jnp.float32)
    pc = (zc > 0.5).astype(jnp.float32)
    csize = jnp.sum(pr)
    ppair = pr * pc

    cm = conf_ref[...]
    omega = 0.5 * (cm + cm.T)
    omega = jnp.where(eye, 0.0, omega)

    pairmask = ppair * (row_i > col_i).astype(jnp.float32)
    contrib = pairmask * omega * jnp.exp(-_ALPHA * d2)
    num_all = jnp.sum(contrib)

    scores = jnp.where(ppair > 0.5, omega, -1.0)
    si = jnp.round(jnp.maximum(scores, 0.0) * 16777216.0).astype(jnp.int32)
    si = jnp.where(scores < 0.0, jnp.int32(-1), si)
    key_ref[...] = si * 1024 + (1023 - row_i)

    taken = jnp.int32(-(2 ** 31 - 1))

    def tk_body(_, carry):
        k = key_ref[...]
        mx = jnp.max(k, axis=0, keepdims=True)
        key_ref[...] = jnp.where(k == mx, taken, k)
        return carry

    jax.lax.fori_loop(0, _TOPM, tk_body, jnp.int32(0))
    mskf = (key_ref[...] == taken).astype(jnp.float32)
    num_tk = jnp.sum(mskf * contrib)
    den_tk = jnp.sum(mskf * pairmask)

    loss_all = num_all / jnp.maximum(csize * (csize - 1.0) * 0.5, 1.0)
    loss_tk = num_tk / jnp.maximum(den_tk, 1.0)
    loss = jnp.where(csize < 1.5, 0.0,
                     jnp.where(csize <= float(_TOPM + 1) + 0.5,
                               loss_all, loss_tk))
    out_ref[...] = jnp.broadcast_to(loss, (1, 1))


@jax.jit
def kernel(feats, y, conf):
    y2 = y.astype(jnp.int32).reshape(_B // _CH, _CH)
    feats3 = feats.reshape(_B, 2, _DH)
    zeros = jnp.zeros((_KP, 1, _DH), jnp.float32)
    conf_p = jnp.pad(conf, ((0, _KP - _K), (0, _KP - _K)))

    sums3 = _sc_segsum(y2, feats3, zeros)
    sums = sums3.reshape(_KP, _D)

    out = pl.pallas_call(
        _loss_body,
        out_shape=jax.ShapeDtypeStruct((1, 1), jnp.float32),
        scratch_shapes=[
            pltpu.VMEM((_KP, _KP), jnp.int32),
        ],
    )(sums, conf_p)
    return out[0, 0]


# speedup vs baseline: 3.1161x; 3.1161x over previous
import jax
import jax.numpy as jnp
from jax.experimental import pallas as pl
from jax.experimental.pallas import tpu as pltpu

_K = 1000
_KP = 1024
_D = 2048
_ALPHA = 10.0
_TOPM = 20
_BR = 2048
_NB = 16384 // _BR


_TAKEN = -(2 ** 31 - 1)


def _tk_body(_, carry, key_ref):
    k = key_ref[...]
    mx = jnp.max(k, axis=0, keepdims=True)
    key_ref[...] = jnp.where(k == mx, jnp.int32(_TAKEN), k)
    return carry


def _body(y_ref, f_ref, conf_ref, out_ref, sums_ref, key_ref):
    i = pl.program_id(0)

    @pl.when(i == 0)
    def _build_keys():
        row_i = jax.lax.broadcasted_iota(jnp.int32, (_KP, _KP), 0)
        col_i = jax.lax.broadcasted_iota(jnp.int32, (_KP, _KP), 1)
        cm = conf_ref[...]
        omega = 0.5 * (cm + cm.T)
        omega = jnp.where(row_i == col_i, 0.0, omega)
        si = jnp.round(omega * 16777216.0).astype(jnp.int32)
        key_ref[...] = si * 1024 + (1023 - row_i)

    @pl.when(i < _NB)
    def _segsum():
        yv = y_ref[0]
        rows = jax.lax.broadcasted_iota(jnp.int32, (_KP, _BR), 0)
        oh = (rows == yv).astype(jnp.float32)
        part = jax.lax.dot_general(
            oh, f_ref[...],
            (((1,), (0,)), ((), ())),
            precision=jax.lax.Precision.DEFAULT,
            preferred_element_type=jnp.float32)

        @pl.when(i == 0)
        def _init():
            sums_ref[...] = part.astype(jnp.bfloat16)

        @pl.when(i > 0)
        def _acc():
            sums_ref[...] = (sums_ref[...].astype(jnp.float32)
                             + part).astype(jnp.bfloat16)

    @pl.when((i >= 1) & (i < _NB))
    def _tk_partial():
        jax.lax.fori_loop(
            0, 2, lambda t, c: _tk_body(t, c, key_ref), jnp.int32(0))

    @pl.when(i == _NB)
    def _loss():
        s = sums_ref[...].astype(jnp.float32)
        n2 = jnp.sum(s * s, axis=1, keepdims=True)
        inv = jnp.where(n2 > 0.0, jax.lax.rsqrt(jnp.maximum(n2, 1e-30)), 0.0)
        sn = (s * inv).astype(jnp.bfloat16)
        g = jax.lax.dot_general(
            sn, sn, (((1,), (1,)), ((), ())),
            preferred_element_type=jnp.float32)

        row_i = jax.lax.broadcasted_iota(jnp.int32, (_KP, _KP), 0)
        col_i = jax.lax.broadcasted_iota(jnp.int32, (_KP, _KP), 1)
        eye = row_i == col_i
        gd = jnp.where(eye, g, 0.0)
        zr = jnp.sum(gd, axis=1, keepdims=True)
        zc = jnp.sum(gd, axis=0, keepdims=True)
        d2 = zr + zc - 2.0 * g

        pr = (zr > 0.5).astype(jnp.float32)
        pc = (zc > 0.5).astype(jnp.float32)
        csize = jnp.sum(pr)
        ppair = pr * pc

        cm = conf_ref[...]
        omega = 0.5 * (cm + cm.T)
        omega = jnp.where(eye, 0.0, omega)

        pairmask = ppair * (row_i > col_i).astype(jnp.float32)
        contrib = pairmask * omega * jnp.exp(-_ALPHA * d2)
        num_all = jnp.sum(contrib)

        jax.lax.fori_loop(
            0, _TOPM - 2 * (_NB - 1),
            lambda t, c: _tk_body(t, c, key_ref), jnp.int32(0))
        mskf = (key_ref[...] == jnp.int32(_TAKEN)).astype(jnp.float32)
        num_tk = jnp.sum(mskf * contrib)
        den_tk = jnp.sum(mskf * pairmask)

        loss_all = num_all / jnp.maximum(csize * (csize - 1.0) * 0.5, 1.0)
        loss_tk = num_tk / jnp.maximum(den_tk, 1.0)
        loss = jnp.where(csize < 1.5, 0.0,
                         jnp.where(csize <= float(_TOPM + 1) + 0.5,
                                   loss_all, loss_tk))
        out_ref[...] = jnp.broadcast_to(loss, (1, 1))


@jax.jit
def kernel(feats, y, conf):
    y3 = y.astype(jnp.int32).reshape(_NB, 1, _BR)
    conf_p = jnp.pad(conf, ((0, _KP - _K), (0, _KP - _K)))

    out = pl.pallas_call(
        _body,
        grid=(_NB + 1,),
        in_specs=[
            pl.BlockSpec((1, 1, _BR), lambda i: (jnp.minimum(i, _NB - 1), 0, 0)),
            pl.BlockSpec((_BR, _D), lambda i: (jnp.minimum(i, _NB - 1), 0)),
            pl.BlockSpec((_KP, _KP), lambda i: (0, 0)),
        ],
        out_specs=pl.BlockSpec((1, 1), lambda i: (0, 0)),
        out_shape=jax.ShapeDtypeStruct((1, 1), jnp.float32),
        scratch_shapes=[
            pltpu.VMEM((_KP, _D), jnp.bfloat16),
            pltpu.VMEM((_KP, _KP), jnp.int32),
        ],
    )(y3, feats, conf_p)
    return out[0, 0]


# speedup vs baseline: 3.1183x; 1.0007x over previous
import jax
import jax.numpy as jnp
from jax.experimental import pallas as pl
from jax.experimental.pallas import tpu as pltpu

_K = 1000
_KP = 1024
_D = 2048
_ALPHA = 10.0
_TOPM = 20
_BR = 2048
_NB = 16384 // _BR


_TAKEN = -(2 ** 31 - 1)


def _tk_body(_, carry, key_ref):
    k = key_ref[...]
    mx = jnp.max(k, axis=0, keepdims=True)
    key_ref[...] = jnp.where(k == mx, jnp.int32(_TAKEN), k)
    return carry


def _body(y_ref, f_ref, conf_ref, out_ref, sums_ref, key_ref):
    i = pl.program_id(0)

    @pl.when(i == 0)
    def _build_keys():
        row_i = jax.lax.broadcasted_iota(jnp.int32, (_KP, _KP), 0)
        col_i = jax.lax.broadcasted_iota(jnp.int32, (_KP, _KP), 1)
        cm = conf_ref[...]
        omega = 0.5 * (cm + cm.T)
        omega = jnp.where(row_i == col_i, 0.0, omega)
        si = jnp.round(omega * 16777216.0).astype(jnp.int32)
        key_ref[...] = si * 1024 + (1023 - row_i)

    @pl.when((i >= 1) & (i <= _NB))
    def _segsum():
        yv = y_ref[0]
        rows = jax.lax.broadcasted_iota(jnp.int32, (_KP, _BR), 0)
        oh = (rows == yv).astype(jnp.float32)
        part = jax.lax.dot_general(
            oh, f_ref[...],
            (((1,), (0,)), ((), ())),
            precision=jax.lax.Precision.DEFAULT,
            preferred_element_type=jnp.float32)

        @pl.when(i == 1)
        def _init():
            sums_ref[...] = part.astype(jnp.bfloat16)

        @pl.when(i > 1)
        def _acc():
            sums_ref[...] = (sums_ref[...].astype(jnp.float32)
                             + part).astype(jnp.bfloat16)

    @pl.when((i >= 2) & (i <= _NB))
    def _tk_partial():
        jax.lax.fori_loop(
            0, 2, lambda t, c: _tk_body(t, c, key_ref), jnp.int32(0))

    @pl.when(i == _NB + 1)
    def _loss():
        s = sums_ref[...].astype(jnp.float32)
        n2 = jnp.sum(s * s, axis=1, keepdims=True)
        inv = jnp.where(n2 > 0.0, jax.lax.rsqrt(jnp.maximum(n2, 1e-30)), 0.0)
        sn = (s * inv).astype(jnp.bfloat16)
        g = jax.lax.dot_general(
            sn, sn, (((1,), (1,)), ((), ())),
            preferred_element_type=jnp.float32)

        row_i = jax.lax.broadcasted_iota(jnp.int32, (_KP, _KP), 0)
        col_i = jax.lax.broadcasted_iota(jnp.int32, (_KP, _KP), 1)
        eye = row_i == col_i
        gd = jnp.where(eye, g, 0.0)
        zr = jnp.sum(gd, axis=1, keepdims=True)
        zc = jnp.sum(gd, axis=0, keepdims=True)
        d2 = zr + zc - 2.0 * g

        pr = (zr > 0.5).astype(jnp.float32)
        pc = (zc > 0.5).astype(jnp.float32)
        csize = jnp.sum(pr)
        ppair = pr * pc

        cm = conf_ref[...]
        omega = 0.5 * (cm + cm.T)
        omega = jnp.where(eye, 0.0, omega)

        pairmask = ppair * (row_i > col_i).astype(jnp.float32)
        contrib = pairmask * omega * jnp.exp(-_ALPHA * d2)
        num_all = jnp.sum(contrib)

        jax.lax.fori_loop(
            0, _TOPM - 2 * (_NB - 1),
            lambda t, c: _tk_body(t, c, key_ref), jnp.int32(0))
        mskf = (key_ref[...] == jnp.int32(_TAKEN)).astype(jnp.float32)
        num_tk = jnp.sum(mskf * contrib)
        den_tk = jnp.sum(mskf * pairmask)

        loss_all = num_all / jnp.maximum(csize * (csize - 1.0) * 0.5, 1.0)
        loss_tk = num_tk / jnp.maximum(den_tk, 1.0)
        loss = jnp.where(csize < 1.5, 0.0,
                         jnp.where(csize <= float(_TOPM + 1) + 0.5,
                                   loss_all, loss_tk))
        out_ref[...] = jnp.broadcast_to(loss, (1, 1))


@jax.jit
def kernel(feats, y, conf):
    y3 = y.astype(jnp.int32).reshape(_NB, 1, _BR)
    conf_p = jnp.pad(conf, ((0, _KP - _K), (0, _KP - _K)))

    def _blk(i):
        return jnp.clip(i - 1, 0, _NB - 1)

    out = pl.pallas_call(
        _body,
        grid=(_NB + 2,),
        in_specs=[
            pl.BlockSpec((1, 1, _BR), lambda i: (_blk(i), 0, 0)),
            pl.BlockSpec((_BR, _D), lambda i: (_blk(i), 0)),
            pl.BlockSpec((_KP, _KP), lambda i: (0, 0)),
        ],
        out_specs=pl.BlockSpec((1, 1), lambda i: (0, 0)),
        out_shape=jax.ShapeDtypeStruct((1, 1), jnp.float32),
        scratch_shapes=[
            pltpu.VMEM((_KP, _D), jnp.bfloat16),
            pltpu.VMEM((_KP, _KP), jnp.int32),
        ],
    )(y3, feats, conf_p)
    return out[0, 0]


# speedup vs baseline: 3.1240x; 1.0018x over previous
import jax
import jax.numpy as jnp
from jax.experimental import pallas as pl
from jax.experimental.pallas import tpu as pltpu

_K = 1000
_KP = 1024
_D = 2048
_ALPHA = 10.0
_TOPM = 20
_BR = 2048
_NB = 16384 // _BR


def _body(y_ref, f_ref, conf_ref, out_ref, sums_ref, key_ref):
    i = pl.program_id(0)

    @pl.when(i < _NB)
    def _segsum():
        yv = y_ref[0]
        rows = jax.lax.broadcasted_iota(jnp.int32, (_KP, _BR), 0)
        oh = (rows == yv).astype(jnp.float32)
        part = jax.lax.dot_general(
            oh, f_ref[...],
            (((1,), (0,)), ((), ())),
            precision=jax.lax.Precision.DEFAULT,
            preferred_element_type=jnp.float32)

        @pl.when(i == 0)
        def _init():
            sums_ref[...] = part.astype(jnp.bfloat16)

        @pl.when(i > 0)
        def _acc():
            sums_ref[...] = (sums_ref[...].astype(jnp.float32)
                             + part).astype(jnp.bfloat16)

    @pl.when(i == _NB)
    def _loss():
        s = sums_ref[...].astype(jnp.float32)
        n2 = jnp.sum(s * s, axis=1, keepdims=True)
        inv = jnp.where(n2 > 0.0, jax.lax.rsqrt(jnp.maximum(n2, 1e-30)), 0.0)
        sn = (s * inv).astype(jnp.bfloat16)
        g = jax.lax.dot_general(
            sn, sn, (((1,), (1,)), ((), ())),
            preferred_element_type=jnp.float32)

        row_i = jax.lax.broadcasted_iota(jnp.int32, (_KP, _KP), 0)
        col_i = jax.lax.broadcasted_iota(jnp.int32, (_KP, _KP), 1)
        eye = row_i == col_i
        gd = jnp.where(eye, g, 0.0)
        zr = jnp.sum(gd, axis=1, keepdims=True)
        zc = jnp.sum(gd, axis=0, keepdims=True)
        d2 = zr + zc - 2.0 * g

        pr = (zr > 0.5).astype(jnp.float32)
        pc = (zc > 0.5).astype(jnp.float32)
        csize = jnp.sum(pr)
        ppair = pr * pc

        cm = conf_ref[...]
        omega = 0.5 * (cm + cm.T)
        omega = jnp.where(eye, 0.0, omega)

        pairmask = ppair * (row_i > col_i).astype(jnp.float32)
        contrib = pairmask * omega * jnp.exp(-_ALPHA * d2)
        num_all = jnp.sum(contrib)

        scores = jnp.where(ppair > 0.5, omega, -1.0)
        si = jnp.round(jnp.maximum(scores, 0.0) * 16777216.0).astype(jnp.int32)
        si = jnp.where(scores < 0.0, jnp.int32(-1), si)
        key_ref[...] = si * 1024 + (1023 - row_i)

        taken = jnp.int32(-(2 ** 31 - 1))

        def tk_body(_, carry):
            k = key_ref[...]
            mx = jnp.max(k, axis=0, keepdims=True)
            key_ref[...] = jnp.where(k == mx, taken, k)
            return carry

        jax.lax.fori_loop(0, _TOPM, tk_body, jnp.int32(0))
        mskf = (key_ref[...] == taken).astype(jnp.float32)
        num_tk = jnp.sum(mskf * contrib)
        den_tk = jnp.sum(mskf * pairmask)

        loss_all = num_all / jnp.maximum(csize * (csize - 1.0) * 0.5, 1.0)
        loss_tk = num_tk / jnp.maximum(den_tk, 1.0)
        loss = jnp.where(csize < 1.5, 0.0,
                         jnp.where(csize <= float(_TOPM + 1) + 0.5,
                                   loss_all, loss_tk))
        out_ref[...] = jnp.broadcast_to(loss, (1, 1))


@jax.jit
def kernel(feats, y, conf):
    y3 = y.astype(jnp.int32).reshape(_NB, 1, _BR)
    conf_p = jnp.pad(conf, ((0, _KP - _K), (0, _KP - _K)))

    out = pl.pallas_call(
        _body,
        grid=(_NB + 1,),
        in_specs=[
            pl.BlockSpec((1, 1, _BR), lambda i: (jnp.minimum(i, _NB - 1), 0, 0)),
            pl.BlockSpec((_BR, _D), lambda i: (jnp.minimum(i, _NB - 1), 0)),
            pl.BlockSpec((_KP, _KP), lambda i: (0, 0)),
        ],
        out_specs=pl.BlockSpec((1, 1), lambda i: (0, 0)),
        out_shape=jax.ShapeDtypeStruct((1, 1), jnp.float32),
        scratch_shapes=[
            pltpu.VMEM((_KP, _D), jnp.bfloat16),
            pltpu.VMEM((_KP, _KP), jnp.int32),
        ],
    )(y3, feats, conf_p)
    return out[0, 0]


# speedup vs baseline: 3.2552x; 1.0420x over previous
import jax
import jax.numpy as jnp
from jax.experimental import pallas as pl
from jax.experimental.pallas import tpu as pltpu

_K = 1000
_KP = 1024
_D = 2048
_ALPHA = 10.0
_TOPM = 20
_BR = 2048
_NB = 16384 // _BR


def _body(y_ref, f_ref, conf_ref, out_ref, sums_ref, key_ref):
    i = pl.program_id(0)

    @pl.when(i < _NB)
    def _segsum():
        yv = y_ref[0]
        rows = jax.lax.broadcasted_iota(jnp.int32, (_KP, _BR), 0)
        oh = (rows == yv).astype(jnp.float32)
        part = jax.lax.dot_general(
            oh, f_ref[...],
            (((1,), (0,)), ((), ())),
            precision=jax.lax.Precision.DEFAULT,
            preferred_element_type=jnp.float32)

        @pl.when(i == 0)
        def _init():
            sums_ref[...] = part.astype(jnp.bfloat16)

        @pl.when(i > 0)
        def _acc():
            sums_ref[...] = (sums_ref[...].astype(jnp.float32)
                             + part).astype(jnp.bfloat16)

    @pl.when(i == _NB)
    def _loss():
        s = sums_ref[...].astype(jnp.float32)
        n2 = jnp.sum(s * s, axis=1, keepdims=True)
        inv = jnp.where(n2 > 0.0, jax.lax.rsqrt(jnp.maximum(n2, 1e-30)), 0.0)
        sn = (s * inv).astype(jnp.bfloat16)
        g = jax.lax.dot_general(
            sn, sn, (((1,), (1,)), ((), ())),
            preferred_element_type=jnp.float32)

        row_i = jax.lax.broadcasted_iota(jnp.int32, (_KP, _KP), 0)
        col_i = jax.lax.broadcasted_iota(jnp.int32, (_KP, _KP), 1)
        eye = row_i == col_i
        gd = jnp.where(eye, g, 0.0)
        zr = jnp.sum(gd, axis=1, keepdims=True)
        zc = jnp.sum(gd, axis=0, keepdims=True)
        d2 = zr + zc - 2.0 * g

        pr = (zr > 0.5).astype(jnp.float32)
        pc = (zc > 0.5).astype(jnp.float32)
        csize = jnp.sum(pr)
        ppair = pr * pc

        cm = jnp.pad(conf_ref[...], ((0, _KP - _K), (0, _KP - _K)))
        omega = 0.5 * (cm + cm.T)
        omega = jnp.where(eye, 0.0, omega)

        pairmask = ppair * (row_i > col_i).astype(jnp.float32)
        contrib = pairmask * omega * jnp.exp(-_ALPHA * d2)
        num_all = jnp.sum(contrib)

        scores = jnp.where(ppair > 0.5, omega, -1.0)
        si = jnp.round(jnp.maximum(scores, 0.0) * 16777216.0).astype(jnp.int32)
        si = jnp.where(scores < 0.0, jnp.int32(-1), si)
        key_ref[...] = si * 1024 + (1023 - row_i)

        taken = jnp.int32(-(2 ** 31 - 1))

        def tk_body(_, carry):
            k = key_ref[...]
            mx = jnp.max(k, axis=0, keepdims=True)
            key_ref[...] = jnp.where(k == mx, taken, k)
            return carry

        jax.lax.fori_loop(0, _TOPM, tk_body, jnp.int32(0))
        mskf = (key_ref[...] == taken).astype(jnp.float32)
        num_tk = jnp.sum(mskf * contrib)
        den_tk = jnp.sum(mskf * pairmask)

        loss_all = num_all / jnp.maximum(csize * (csize - 1.0) * 0.5, 1.0)
        loss_tk = num_tk / jnp.maximum(den_tk, 1.0)
        loss = jnp.where(csize < 1.5, 0.0,
                         jnp.where(csize <= float(_TOPM + 1) + 0.5,
                                   loss_all, loss_tk))
        out_ref[...] = jnp.broadcast_to(loss, (1, 1))


@jax.jit
def kernel(feats, y, conf):
    y3 = y.astype(jnp.int32).reshape(_NB, 1, _BR)

    out = pl.pallas_call(
        _body,
        grid=(_NB + 1,),
        in_specs=[
            pl.BlockSpec((1, 1, _BR), lambda i: (jnp.minimum(i, _NB - 1), 0, 0)),
            pl.BlockSpec((_BR, _D), lambda i: (jnp.minimum(i, _NB - 1), 0)),
            pl.BlockSpec((_K, _K), lambda i: (0, 0)),
        ],
        out_specs=pl.BlockSpec((1, 1), lambda i: (0, 0)),
        out_shape=jax.ShapeDtypeStruct((1, 1), jnp.float32),
        scratch_shapes=[
            pltpu.VMEM((_KP, _D), jnp.bfloat16),
            pltpu.VMEM((_KP, _KP), jnp.int32),
        ],
    )(y3, feats, conf)
    return out[0, 0]
